# baseline (speedup 1.0000x reference)
import functools

import jax
import jax.numpy as jnp
from jax.experimental import pallas as pl
from jax.experimental.pallas import tpu as pltpu

B, C, L = 16, 8, 65536
ROWS = 8
PAD = 128
CH = 512
NC = L // CH
TREND_K = 11
LOCAL_K = 5
TREND_H = 5
LOCAL_H = 2
TREND_SCALING = 0.6
DETAIL = 0.85
SPIKE_T = 3.5
SPIKE_D = 0.35
EPS = 1e-6
ITERS = 2


def _edge_matrices(h, k):
    i = jax.lax.broadcasted_iota(jnp.int32, (256, 128), 0)
    p = jax.lax.broadcasted_iota(jnp.int32, (256, 128), 1)
    inv_k = 1.0 / float(k)
    left = ((jnp.abs(i - p) <= h).astype(jnp.float32)
            + ((i >= 1) & (i <= h - p)).astype(jnp.float32)) * inv_k
    q = 128 + p
    right = ((jnp.abs(i - q) <= h).astype(jnp.float32)
             + ((i >= 510 - q - h) & (i <= 254)).astype(jnp.float32)) * inv_k
    return left, right


_dot = functools.partial(
    jax.lax.dot_general,
    dimension_numbers=(((1,), (0,)), ((), ())),
    preferred_element_type=jnp.float32,
    precision=jax.lax.Precision.HIGHEST)


def _denoise_body(x_ref, o_ref, ps_ref, rs_ref, bs_ref):
    t5l, t5r = _edge_matrices(TREND_H, TREND_K)
    l5l, l5r = _edge_matrices(LOCAL_H, LOCAL_K)

    ps_ref[:, 0:PAD] = jnp.zeros((ROWS, PAD), dtype=jnp.float32)
    ps_ref[:, PAD + L:PAD + L + PAD] = jnp.zeros((ROWS, PAD), dtype=jnp.float32)
    ps_ref[:, PAD:PAD + L] = x_ref[...]

    def one_iter(dst_ref, dst_base):
        def conv_chunk(c, _):
            base = pl.multiple_of(PAD - 128 + c * CH, 128)
            halo = ps_ref[:, pl.ds(base, CH + 256)]
            t = {j: halo[:, 128 + j:128 + j + CH] for j in range(-5, 6)}
            s2 = ((t[-2] + t[-1]) + (t[0] + t[1])) + t[2]
            s5 = s2 + (((t[-5] + t[-4]) + (t[-3] + t[3])) + (t[4] + t[5]))
            local = s2 * (1.0 / LOCAL_K)
            trend = s5 * (1.0 / TREND_K)
            ob = pl.multiple_of(c * CH, 128)
            rs_ref[:, pl.ds(ob, CH)] = t[0] - local
            bs_ref[:, pl.ds(ob, CH)] = (
                (1.0 - TREND_SCALING) * local + TREND_SCALING * trend)
            return 0

        jax.lax.fori_loop(0, NC, conv_chunk, 0, unroll=2)

        xl = ps_ref[:, PAD:PAD + 256]
        xr = ps_ref[:, PAD + L - 256:PAD + L]
        lL, tL = _dot(xl, l5l), _dot(xl, t5l)
        lR, tR = _dot(xr, l5r), _dot(xr, t5r)
        rs_ref[:, 0:128] = xl[:, :128] - lL
        bs_ref[:, 0:128] = (1.0 - TREND_SCALING) * lL + TREND_SCALING * tL
        rs_ref[:, L - 128:L] = xr[:, 128:] - lR
        bs_ref[:, L - 128:L] = (1.0 - TREND_SCALING) * lR + TREND_SCALING * tR

        def stat_chunk(c, acc):
            acc_s, acc_q = acc
            r = rs_ref[:, pl.ds(pl.multiple_of(c * CH, 128), CH)]
            return acc_s + r, acc_q + r * r

        zero = jnp.zeros((ROWS, CH), dtype=jnp.float32)
        acc_s, acc_q = jax.lax.fori_loop(
            0, NC, stat_chunk, (zero, zero), unroll=2)
        sum_r = jnp.sum(acc_s, axis=1, keepdims=True)
        sum_q = jnp.sum(acc_q, axis=1, keepdims=True)
        var = (sum_q - sum_r * sum_r * (1.0 / L)) * (1.0 / (L - 1))
        scale = jnp.maximum(jnp.sqrt(jnp.maximum(var, 0.0)), EPS)
        thr = scale * SPIKE_T

        def out_chunk(c, _):
            ob = pl.multiple_of(c * CH, 128)
            r = rs_ref[:, pl.ds(ob, CH)]
            bl = bs_ref[:, pl.ds(ob, CH)]
            rc = jnp.where(jnp.abs(r) > thr, r * (DETAIL * SPIKE_D), r * DETAIL)
            dst_ref[:, pl.ds(pl.multiple_of(dst_base + c * CH, 128), CH)] = bl + rc
            return 0

        jax.lax.fori_loop(0, NC, out_chunk, 0, unroll=2)

    for it in range(ITERS):
        if it < ITERS - 1:
            one_iter(ps_ref, PAD)
        else:
            one_iter(o_ref, 0)


@jax.jit
def kernel(x):
    xf = x.astype(jnp.float32).reshape(B * C, L)
    out = pl.pallas_call(
        _denoise_body,
        grid=(B * C // ROWS,),
        in_specs=[pl.BlockSpec((ROWS, L), lambda i: (i, 0))],
        out_specs=pl.BlockSpec((ROWS, L), lambda i: (i, 0)),
        out_shape=jax.ShapeDtypeStruct((B * C, L), jnp.float32),
        scratch_shapes=[pltpu.VMEM((ROWS, L + 2 * PAD), jnp.float32),
                        pltpu.VMEM((ROWS, L), jnp.float32),
                        pltpu.VMEM((ROWS, L), jnp.float32)],
        compiler_params=pltpu.CompilerParams(
            dimension_semantics=("parallel",)),
    )(xf)
    return out.reshape(B, C, L)


# speedup vs baseline: 3.5568x; 3.5568x over previous
import functools

import jax
import jax.numpy as jnp
from jax.experimental import pallas as pl
from jax.experimental.pallas import tpu as pltpu

B, C, L = 16, 8, 65536
ROWS = 8
PAD = 128
TREND_K = 11
LOCAL_K = 5
TREND_H = 5
LOCAL_H = 2
TREND_SCALING = 0.6
DETAIL = 0.85
SPIKE_T = 3.5
SPIKE_D = 0.35
EPS = 1e-6
ITERS = 2


def _edge_matrices(h, k):
    i = jax.lax.broadcasted_iota(jnp.int32, (256, 128), 0)
    p = jax.lax.broadcasted_iota(jnp.int32, (256, 128), 1)
    inv_k = 1.0 / float(k)
    left = ((jnp.abs(i - p) <= h).astype(jnp.float32)
            + ((i >= 1) & (i <= h - p)).astype(jnp.float32)) * inv_k
    q = 128 + p
    right = ((jnp.abs(i - q) <= h).astype(jnp.float32)
             + ((i >= 510 - q - h) & (i <= 254)).astype(jnp.float32)) * inv_k
    return left, right


_dot = functools.partial(
    jax.lax.dot_general,
    dimension_numbers=(((1,), (0,)), ((), ())),
    preferred_element_type=jnp.float32,
    precision=jax.lax.Precision.HIGHEST)


def _denoise_body(x_ref, o_ref, ps_ref, s2_ref):
    l5l, l5r = _edge_matrices(LOCAL_H, LOCAL_K)
    t11l, t11r = _edge_matrices(TREND_H, TREND_K)
    bl_l = (1.0 - TREND_SCALING) * l5l + TREND_SCALING * t11l
    bl_r = (1.0 - TREND_SCALING) * l5r + TREND_SCALING * t11r

    zpad = jnp.zeros((ROWS, PAD), dtype=jnp.float32)
    ps_ref[:, 0:PAD] = zpad
    ps_ref[:, PAD + L:PAD + L + PAD] = zpad
    s2_ref[:, 0:PAD] = zpad
    s2_ref[:, PAD + L:PAD + L + PAD] = zpad

    def one_iter(v):
        ps_ref[:, PAD:PAD + L] = v
        t = {j: ps_ref[:, PAD + j:PAD + j + L] for j in (-2, -1, 0, 1, 2)}
        s2 = ((t[-2] + t[-1]) + (t[1] + t[2])) + t[0]
        s2_ref[:, PAD:PAD + L] = s2
        s5 = (s2_ref[:, PAD - 3:PAD - 3 + L]
              + s2_ref[:, PAD + 3:PAD + 3 + L]) + t[0]
        rm = v - s2 * (1.0 / LOCAL_K)
        bm = (s2 * ((1.0 - TREND_SCALING) / LOCAL_K)
              + s5 * (TREND_SCALING / TREND_K))

        xl = v[:, :256]
        xr = v[:, L - 256:]
        r = jnp.concatenate(
            [xl[:, :128] - _dot(xl, l5l), rm[:, 128:L - 128],
             xr[:, 128:] - _dot(xr, l5r)], axis=1)
        blended = jnp.concatenate(
            [_dot(xl, bl_l), bm[:, 128:L - 128], _dot(xr, bl_r)], axis=1)

        sum_r = jnp.sum(r, axis=1, keepdims=True)
        sum_q = jnp.sum(r * r, axis=1, keepdims=True)
        var = (sum_q - sum_r * sum_r * (1.0 / L)) * (1.0 / (L - 1))
        scale = jnp.maximum(jnp.sqrt(jnp.maximum(var, 0.0)), EPS)
        thr = scale * SPIKE_T
        rc = jnp.where(jnp.abs(r) > thr, r * (DETAIL * SPIKE_D), r * DETAIL)
        return blended + rc

    v = x_ref[...]
    for _ in range(ITERS):
        v = one_iter(v)
    o_ref[...] = v


@jax.jit
def kernel(x):
    xf = x.astype(jnp.float32).reshape(B * C, L)
    out = pl.pallas_call(
        _denoise_body,
        grid=(B * C // ROWS,),
        in_specs=[pl.BlockSpec((ROWS, L), lambda i: (i, 0))],
        out_specs=pl.BlockSpec((ROWS, L), lambda i: (i, 0)),
        out_shape=jax.ShapeDtypeStruct((B * C, L), jnp.float32),
        scratch_shapes=[pltpu.VMEM((ROWS, L + 2 * PAD), jnp.float32),
                        pltpu.VMEM((ROWS, L + 2 * PAD), jnp.float32)],
        compiler_params=pltpu.CompilerParams(
            dimension_semantics=("parallel",)),
    )(xf)
    return out.reshape(B, C, L)


# speedup vs baseline: 3.9968x; 1.1237x over previous
import functools

import jax
import jax.numpy as jnp
from jax.experimental import pallas as pl
from jax.experimental.pallas import tpu as pltpu

B, C, L = 16, 8, 65536
ROWS = 8
PAD = 128
TREND_K = 11
LOCAL_K = 5
TREND_H = 5
LOCAL_H = 2
TREND_SCALING = 0.6
DETAIL = 0.85
SPIKE_T = 3.5
SPIKE_D = 0.35
EPS = 1e-6
ITERS = 2


def _edge_matrices(h, k):
    i = jax.lax.broadcasted_iota(jnp.int32, (256, 128), 0)
    p = jax.lax.broadcasted_iota(jnp.int32, (256, 128), 1)
    inv_k = 1.0 / float(k)
    left = ((jnp.abs(i - p) <= h).astype(jnp.float32)
            + ((i >= 1) & (i <= h - p)).astype(jnp.float32)) * inv_k
    q = 128 + p
    right = ((jnp.abs(i - q) <= h).astype(jnp.float32)
             + ((i >= 510 - q - h) & (i <= 254)).astype(jnp.float32)) * inv_k
    return left, right


_dot = functools.partial(
    jax.lax.dot_general,
    dimension_numbers=(((1,), (0,)), ((), ())),
    preferred_element_type=jnp.float32,
    precision=jax.lax.Precision.HIGHEST)


def _denoise_body(x_ref, o_ref, ps_ref, w2_ref, s2_ref, rs_ref, bs_ref):
    l5l, l5r = _edge_matrices(LOCAL_H, LOCAL_K)
    t11l, t11r = _edge_matrices(TREND_H, TREND_K)
    bl_l = (1.0 - TREND_SCALING) * l5l + TREND_SCALING * t11l
    bl_r = (1.0 - TREND_SCALING) * l5r + TREND_SCALING * t11r

    zpad = jnp.zeros((ROWS, PAD), dtype=jnp.float32)
    for ref in (ps_ref, w2_ref, s2_ref):
        ref[:, 0:PAD] = zpad
        ref[:, PAD + L:PAD + L + PAD] = zpad

    def one_iter(dst_ref, dst_base):
        t0 = ps_ref[:, PAD:PAD + L]
        w2_ref[:, PAD:PAD + L] = t0 + ps_ref[:, PAD + 1:PAD + 1 + L]
        s2 = ((w2_ref[:, PAD - 2:PAD - 2 + L] + w2_ref[:, PAD:PAD + L])
              + ps_ref[:, PAD + 2:PAD + 2 + L])
        s2_ref[:, PAD:PAD + L] = s2
        s5 = (s2_ref[:, PAD - 3:PAD - 3 + L]
              + s2_ref[:, PAD + 3:PAD + 3 + L]) + t0
        rs_ref[:, 0:L] = t0 - s2 * (1.0 / LOCAL_K)
        bs_ref[:, 0:L] = (s2 * ((1.0 - TREND_SCALING) / LOCAL_K)
                          + s5 * (TREND_SCALING / TREND_K))

        xl = ps_ref[:, PAD:PAD + 256]
        xr = ps_ref[:, PAD + L - 256:PAD + L]
        rs_ref[:, 0:128] = xl[:, :128] - _dot(xl, l5l)
        bs_ref[:, 0:128] = _dot(xl, bl_l)
        rs_ref[:, L - 128:L] = xr[:, 128:] - _dot(xr, l5r)
        bs_ref[:, L - 128:L] = _dot(xr, bl_r)

        r = rs_ref[:, 0:L]
        rsq = r * r
        sum_r = jnp.sum(r, axis=1, keepdims=True)
        sum_q = jnp.sum(rsq, axis=1, keepdims=True)
        var = (sum_q - sum_r * sum_r * (1.0 / L)) * (1.0 / (L - 1))
        scale = jnp.maximum(jnp.sqrt(jnp.maximum(var, 0.0)), EPS)
        thr2 = (scale * scale) * (SPIKE_T * SPIKE_T)
        rc = jnp.where(rsq > thr2, r * (DETAIL * SPIKE_D), r * DETAIL)
        dst_ref[:, dst_base:dst_base + L] = bs_ref[:, 0:L] + rc

    ps_ref[:, PAD:PAD + L] = x_ref[...]
    for it in range(ITERS):
        if it < ITERS - 1:
            one_iter(ps_ref, PAD)
        else:
            one_iter(o_ref, 0)


@jax.jit
def kernel(x):
    xf = x.astype(jnp.float32).reshape(B * C, L)
    out = pl.pallas_call(
        _denoise_body,
        grid=(B * C // ROWS,),
        in_specs=[pl.BlockSpec((ROWS, L), lambda i: (i, 0))],
        out_specs=pl.BlockSpec((ROWS, L), lambda i: (i, 0)),
        out_shape=jax.ShapeDtypeStruct((B * C, L), jnp.float32),
        scratch_shapes=[pltpu.VMEM((ROWS, L + 2 * PAD), jnp.float32),
                        pltpu.VMEM((ROWS, L + 2 * PAD), jnp.float32),
                        pltpu.VMEM((ROWS, L + 2 * PAD), jnp.float32),
                        pltpu.VMEM((ROWS, L), jnp.float32),
                        pltpu.VMEM((ROWS, L), jnp.float32)],
        compiler_params=pltpu.CompilerParams(
            dimension_semantics=("parallel",)),
    )(xf)
    return out.reshape(B, C, L)
